# baseline (speedup 1.0000x reference)
import dataclasses
import numpy as np
import jax
import jax.numpy as jnp
from jax import lax
from jax.experimental import pallas as pl
from jax.experimental.pallas import tpu as pltpu
from jax.experimental.pallas import tpu_sc as plsc

_FROM = (0, 1, 2, 3, 4, 5, 6, 3, 8, 9, 10, 3, 12, 13, 14, 0, 16, 17, 18, 0, 20, 21)
_TO = tuple(range(1, 23))
_NUM_LIMBS = 22

_NC, _NS, _L = 2, 16, 16
_NW = _NC * _NS


def _sqrt16(x):
    x = jnp.maximum(x, jnp.float32(1e-35))
    xi = lax.bitcast_convert_type(x, jnp.int32)
    yi = jnp.int32(0x5F3759DF) - lax.shift_right_logical(xi, 1)
    y = lax.bitcast_convert_type(yi, jnp.float32)
    xh = x * jnp.float32(0.5)
    for _ in range(3):
        y = y * (jnp.float32(1.5) - xh * y * y)
    return x * y


def kernel(kpts_gt, kpts_pred):
    n, ncoord, nkpt = kpts_gt.shape
    bpw = n // _NW
    ngroups = bpw // _L

    mesh = plsc.VectorSubcoreMesh(core_axis_name="c", subcore_axis_name="s")
    cp = pltpu.CompilerParams()
    if "needs_layout_passes" in pltpu.CompilerParams.__dataclass_fields__:
        cp = dataclasses.replace(cp, needs_layout_passes=False)

    nfeat = ncoord * nkpt
    wpw = bpw * nfeat

    @jax.jit
    def run(gt, pr):
        @pl.kernel(
            out_type=jax.ShapeDtypeStruct((_NW, _L), jnp.float32),
            mesh=mesh,
            compiler_params=cp,
            scratch_types=[
                pltpu.VMEM((wpw,), jnp.float32),
                pltpu.VMEM((wpw,), jnp.float32),
                pltpu.VMEM((_L,), jnp.float32),
            ],
        )
        def sc_loss(gt_hbm, pr_hbm, out_hbm, gt_v, pr_v, acc_v):
            cid = lax.axis_index("c")
            sid = lax.axis_index("s")
            wid = sid * _NC + cid
            base = wid * wpw
            pltpu.sync_copy(gt_hbm.at[pl.ds(base, wpw)], gt_v)
            pltpu.sync_copy(pr_hbm.at[pl.ds(base, wpw)], pr_v)
            acc_v[...] = jnp.zeros((_L,), jnp.float32)
            viota = lax.iota(jnp.int32, _L) * nfeat

            @pl.loop(0, ngroups)
            def _(g):
                vb = viota + g * (_L * nfeat)
                accs = [None] * _NUM_LIMBS
                for c in range(ncoord):
                    cols = []
                    for k in range(nkpt):
                        idx = vb + (c * nkpt + k)
                        a = plsc.load_gather(gt_v, [idx])
                        b = plsc.load_gather(pr_v, [idx])
                        cols.append(a - b)
                    for l in range(_NUM_LIMBS):
                        t = cols[_FROM[l]] - cols[_TO[l]]
                        sq = t * t
                        accs[l] = sq if accs[l] is None else accs[l] + sq
                total = acc_v[...]
                for l in range(_NUM_LIMBS):
                    total = total + _sqrt16(accs[l])
                acc_v[...] = total

            pltpu.sync_copy(acc_v, out_hbm.at[wid])

        return sc_loss(gt.reshape(-1), pr.reshape(-1))

    partials = run(kpts_gt, kpts_pred)
    return jnp.sum(partials) / np.float32(n * _NUM_LIMBS)


# speedup vs baseline: 4.5103x; 4.5103x over previous
import numpy as np
import jax
import jax.numpy as jnp
from jax.experimental import pallas as pl

_FROM = (0, 1, 2, 3, 4, 5, 6, 3, 8, 9, 10, 3, 12, 13, 14, 0, 16, 17, 18, 0, 20, 21)
_TO = tuple(range(1, 23))
_NUM_LIMBS = 22


def _selection_matrix() -> np.ndarray:
    sel = np.zeros((69, 128), dtype=np.float32)
    for c in range(3):
        for l in range(_NUM_LIMBS):
            sel[c * 23 + _FROM[l], 32 * c + l] += 1.0
            sel[c * 23 + _TO[l], 32 * c + l] -= 1.0
    return sel


def _loss_kernel(gt_ref, pr_ref, sel_ref, out_ref):
    i = pl.program_id(0)
    d = gt_ref[...] - pr_ref[...]
    y = jnp.dot(d, sel_ref[...], preferred_element_type=jnp.float32)
    sq = y * y
    v = sq[:, 0:32] + sq[:, 32:64] + sq[:, 64:96]
    part = jnp.sum(jnp.sqrt(v)).reshape(1, 1)

    @pl.when(i == 0)
    def _():
        out_ref[...] = jnp.zeros((1, 1), jnp.float32)

    out_ref[...] += part


def kernel(kpts_gt, kpts_pred):
    n, ncoord, nkpt = kpts_gt.shape
    nfeat = ncoord * nkpt
    block_b = 2048
    grid = n // block_b
    sel = jnp.asarray(_selection_matrix())
    gt2 = kpts_gt.reshape(n, nfeat)
    pr2 = kpts_pred.reshape(n, nfeat)
    total = pl.pallas_call(
        _loss_kernel,
        grid=(grid,),
        in_specs=[
            pl.BlockSpec((block_b, nfeat), lambda i: (i, 0)),
            pl.BlockSpec((block_b, nfeat), lambda i: (i, 0)),
            pl.BlockSpec((nfeat, 128), lambda i: (0, 0)),
        ],
        out_specs=pl.BlockSpec((1, 1), lambda i: (0, 0)),
        out_shape=jax.ShapeDtypeStruct((1, 1), jnp.float32),
    )(gt2, pr2, sel)
    return total[0, 0] / np.float32(n * _NUM_LIMBS)
